# baseline (speedup 1.0000x reference)
import functools

import jax
import jax.numpy as jnp
from jax import lax
from jax.experimental import pallas as pl
from jax.experimental.pallas import tpu as pltpu
from jax.experimental.pallas import tpu_sc as plsc

VOCAB = 1000
L = 20
B = 1024
ROWS = B * L
NC, NS = 2, 16
NW = NC * NS
RPW = ROWS // NW
CHUNK = 40
NCHUNK = RPW // CHUNK


def _sc_gather(table, idx_flat):
    mesh = plsc.VectorSubcoreMesh(core_axis_name="c", subcore_axis_name="s")

    @functools.partial(
        pl.kernel,
        mesh=mesh,
        out_type=jax.ShapeDtypeStruct((ROWS, VOCAB), jnp.float32),
        scratch_types=[
            pltpu.VMEM((RPW,), jnp.int32),
            pltpu.VMEM((CHUNK, VOCAB), jnp.float32),
            pltpu.VMEM((CHUNK, VOCAB), jnp.float32),
            pltpu.VMEM((CHUNK, VOCAB), jnp.float32),
            pltpu.SemaphoreType.DMA,
            pltpu.SemaphoreType.DMA,
            pltpu.SemaphoreType.DMA,
            pltpu.SemaphoreType.DMA,
            pltpu.SemaphoreType.DMA,
            pltpu.SemaphoreType.DMA,
        ],
        compiler_params=pltpu.CompilerParams(use_tc_tiling_on_sc=False),
    )
    def k(table_hbm, idx_hbm, out_hbm, idx_v,
          buf0, buf1, buf2, g0, g1, g2, s0, s1, s2):
        wid = lax.axis_index("s") * NC + lax.axis_index("c")
        r0 = wid * RPW
        pltpu.sync_copy(idx_hbm.at[pl.ds(r0, RPW)], idx_v)

        bufs = (buf0, buf1, buf2)
        gsems = (g0, g1, g2)
        ssems = (s0, s1, s2)

        def gather(c):
            return pltpu.async_copy(
                table_hbm.at[idx_v.at[pl.ds(c * CHUNK, CHUNK)]],
                bufs[c % 3],
                gsems[c % 3],
            )

        def store(c):
            return pltpu.async_copy(
                bufs[c % 3],
                out_hbm.at[pl.ds(r0 + c * CHUNK, CHUNK)],
                ssems[c % 3],
            )

        gd = [None] * NCHUNK
        sd = [None] * NCHUNK
        gd[0] = gather(0)
        gd[1] = gather(1)
        gd[2] = gather(2)
        for c in range(NCHUNK):
            if c >= 1:
                sd[c - 1].wait()
                if c + 2 < NCHUNK:
                    gd[c + 2] = gather(c + 2)
            gd[c].wait()
            sd[c] = store(c)
        sd[NCHUNK - 1].wait()

    return k(table, idx_flat)


def kernel(inputs, table):
    idx_flat = inputs.astype(jnp.int32).reshape(ROWS)
    return _sc_gather(table, idx_flat).reshape(B, L * VOCAB)


# speedup vs baseline: 1.0031x; 1.0031x over previous
import functools

import jax
import jax.numpy as jnp
from jax import lax
from jax.experimental import pallas as pl
from jax.experimental.pallas import tpu as pltpu
from jax.experimental.pallas import tpu_sc as plsc

VOCAB = 1000
L = 20
B = 1024
ROWS = B * L
NC, NS = 2, 16
NW = NC * NS
RPW = ROWS // NW
CHUNK = 40
NCHUNK = RPW // CHUNK


def _sc_gather(table, idx_flat):
    mesh = plsc.VectorSubcoreMesh(core_axis_name="c", subcore_axis_name="s")

    @functools.partial(
        pl.kernel,
        mesh=mesh,
        out_type=jax.ShapeDtypeStruct((ROWS, VOCAB), jnp.float32),
        scratch_types=[
            pltpu.VMEM((RPW,), jnp.int32),
            pltpu.VMEM((CHUNK, VOCAB), jnp.float32),
            pltpu.VMEM((CHUNK, VOCAB), jnp.float32),
            pltpu.SemaphoreType.DMA,
            pltpu.SemaphoreType.DMA,
            pltpu.SemaphoreType.DMA,
            pltpu.SemaphoreType.DMA,
        ],
        compiler_params=pltpu.CompilerParams(use_tc_tiling_on_sc=False),
    )
    def k(table_hbm, idx_hbm, out_hbm, idx_v, buf0, buf1, g0, g1, s0, s1):
        wid = lax.axis_index("s") * NC + lax.axis_index("c")
        r0 = wid * RPW
        pltpu.sync_copy(idx_hbm.at[pl.ds(r0, RPW)], idx_v)

        bufs = (buf0, buf1)
        gsems = (g0, g1)
        ssems = (s0, s1)

        def gather(c):
            return pltpu.async_copy(
                table_hbm.at[idx_v.at[pl.ds(c * CHUNK, CHUNK)]],
                bufs[c % 2],
                gsems[c % 2],
            )

        def store(c):
            return pltpu.async_copy(
                bufs[c % 2],
                out_hbm.at[pl.ds(r0 + c * CHUNK, CHUNK)],
                ssems[c % 2],
            )

        gd = [None] * NCHUNK
        sd = [None] * NCHUNK
        gd[0] = gather(0)
        gd[1] = gather(1)
        gd[0].wait()
        sd[0] = store(0)
        for c in range(1, NCHUNK):
            sd[c - 1].wait()
            if c + 1 < NCHUNK:
                gd[c + 1] = gather(c + 1)
            gd[c].wait()
            sd[c] = store(c)
        sd[NCHUNK - 1].wait()

    return k(table, idx_flat)


def kernel(inputs, table):
    idx_flat = inputs.astype(jnp.int32).reshape(ROWS)
    return _sc_gather(table, idx_flat).reshape(B, L * VOCAB)
